# baseline (speedup 1.0000x reference)
import functools

import jax
import jax.numpy as jnp
from jax import lax
from jax.experimental import pallas as pl
from jax.experimental.pallas import tpu as pltpu
from jax.experimental.pallas import tpu_sc as plsc

_DIM = 128
_W = 64
_EB = 2048
_G = 128
_ZR = 128


def _ceil_to(x, m):
  return (x + m - 1) // m * m


def _make_seg_sum(N, N_out, E_pad, n_row, has_edge=False, E2P=0):
  assert N % n_row == 0
  RN = N // n_row
  ET = E_pad // 16
  BLKS = ET // _EB
  AR = _ceil_to(RN + 16, 128)
  SLAB = AR // 16
  WS = _ceil_to(-(-RN // 16), 8)
  LAST = RN - 15 * WS
  assert 0 < LAST <= WS and RN % 8 == 0

  mesh = plsc.VectorSubcoreMesh(core_axis_name="c", subcore_axis_name="s",
                                num_cores=2, num_subcores=16)

  scratch = [
      pltpu.VMEM_SHARED((AR, _W), jnp.float32),
      pltpu.VMEM((_EB,), jnp.int32),
      pltpu.VMEM((_EB,), jnp.int32),
      pltpu.VMEM((_EB,), jnp.int32),
      pltpu.VMEM((_EB,), jnp.int32),
      pltpu.VMEM((_G, _W), jnp.float32),
  ]
  if has_edge:
    scratch.append(pltpu.VMEM((_G, _W), jnp.float32))
  scratch += [
      pltpu.VMEM((_ZR, _W), jnp.float32),
      pltpu.SemaphoreType.DMA,
      pltpu.SemaphoreType.DMA,
  ]

  def body(*refs):
    if has_edge:
      (table, eembr, srch, dsth, out,
       acc, sidx, didx, tidx, ldix, rows, erow, zbuf,
       semg, sems) = refs
    else:
      (table, srch, dsth, out,
       acc, sidx, didx, tidx, ldix, rows, zbuf,
       semg, sems) = refs

    cid = lax.axis_index("c")
    sid = lax.axis_index("s")
    base_e = sid * ET

    def zfill(r, carry):
      for k in range(_W // 16):
        zbuf[r, pl.ds(k * 16, 16)] = jnp.zeros((16,), jnp.float32)
      return carry
    lax.fori_loop(0, _ZR, zfill, 0)

    for p in range(n_row):
      lo = p * RN

      off = 0
      while off < SLAB:
        sz = min(_ZR, SLAB - off)
        pltpu.sync_copy(zbuf.at[pl.ds(0, sz)],
                        acc.at[pl.ds(sid * SLAB + off, sz)])
        off += sz
      plsc.subcore_barrier()

      def bbody(b, carry):
        eoff = pl.multiple_of(base_e + b * _EB, _EB)
        pltpu.sync_copy(srch.at[pl.ds(eoff, _EB)], sidx)
        pltpu.sync_copy(dsth.at[pl.ds(eoff, _EB)], didx)

        def sbody(i, carry):
          sv = sidx[pl.ds(i * 16, 16)]
          tidx[pl.ds(i * 16, 16)] = sv * 2 + cid
          dv = didx[pl.ds(i * 16, 16)]
          if n_row == 1:
            ldix[pl.ds(i * 16, 16)] = dv
          else:
            inr = (dv >= lo) & (dv < lo + RN)
            ldix[pl.ds(i * 16, 16)] = jnp.where(
                inr, dv - lo, jnp.int32(RN))
          return carry
        lax.fori_loop(0, _EB // 16, sbody, 0)

        for g in range(_EB // _G):
          pltpu.async_copy(table.at[tidx.at[pl.ds(g * _G, _G)]],
                           rows, semg).wait()
          if has_edge:
            pltpu.sync_copy(
                eembr.at[pl.ds(cid * E2P + eoff + g * _G, _G)], erow)

            def ebody(r, carry):
              for k in range(_W // 16):
                s = pl.ds(k * 16, 16)
                rows[r, s] = jnp.maximum(rows[r, s] + erow[r, s], 0.0)
              return carry
            lax.fori_loop(0, _G, ebody, 0)

          descs = []
          for j in range(_G // 16):
            idxv = ldix[pl.ds(g * _G + j * 16, 16)]
            descs.append(pltpu.async_copy(rows.at[pl.ds(j * 16, 16)],
                                          acc.at[idxv], sems, add=True))
          for d in descs:
            d.wait()
        return carry
      lax.fori_loop(0, BLKS, bbody, 0)

      plsc.subcore_barrier()
      if LAST == WS:
        pltpu.sync_copy(acc.at[pl.ds(sid * WS, WS)],
                        out.at[cid, pl.ds(lo + sid * WS, WS)])
      else:
        @pl.when(sid < 15)
        def _():
          pltpu.sync_copy(acc.at[pl.ds(sid * WS, WS)],
                          out.at[cid, pl.ds(lo + sid * WS, WS)])

        @pl.when(sid == 15)
        def _():
          pltpu.sync_copy(acc.at[pl.ds(15 * WS, LAST)],
                          out.at[cid, pl.ds(lo + 15 * WS, LAST)])
      plsc.subcore_barrier()

  out_type = jax.ShapeDtypeStruct((2, N_out, _W), jnp.float32)
  return pl.kernel(
      body, out_type=out_type, mesh=mesh, scratch_types=scratch,
      compiler_params=pltpu.CompilerParams(use_tc_tiling_on_sc=False))


def _mlp2(xa, xbs, W1, b1, W2, b2):
  N = xa.shape[0]
  BR = 1024
  assert N % BR == 0, N
  ng = len(xbs)
  W = _DIM // ng

  def body(*refs):
    xa_ref = refs[0]
    xb_refs = refs[1:1 + ng]
    w1_ref, b1_ref, w2_ref, b2_ref, o_ref = refs[1 + ng:]
    xb = jnp.concatenate([r[...] for r in xb_refs], axis=1)
    x = xa_ref[...] + xb
    h = jnp.maximum(
        jnp.dot(x, w1_ref[...], preferred_element_type=jnp.float32)
        + b1_ref[...], 0.0)
    o_ref[...] = (
        jnp.dot(h, w2_ref[...], preferred_element_type=jnp.float32)
        + b2_ref[...])

  return pl.pallas_call(
      body,
      grid=(N // BR,),
      in_specs=[pl.BlockSpec((BR, _DIM), lambda i: (i, 0))]
      + [pl.BlockSpec((BR, W), lambda i: (i, 0)) for _ in range(ng)]
      + [
          pl.BlockSpec((_DIM, _DIM), lambda i: (0, 0)),
          pl.BlockSpec((1, _DIM), lambda i: (0, 0)),
          pl.BlockSpec((_DIM, _DIM), lambda i: (0, 0)),
          pl.BlockSpec((1, _DIM), lambda i: (0, 0)),
      ],
      out_specs=pl.BlockSpec((BR, _DIM), lambda i: (i, 0)),
      out_shape=jax.ShapeDtypeStruct((N, _DIM), jnp.float32),
  )(xa, *xbs, W1, b1.reshape(1, _DIM), W2, b2.reshape(1, _DIM))


def _edge_embed(attr, We, be):
  E = attr.shape[0]
  BR = 2048
  assert E % BR == 0
  H = _DIM // 2

  def body(a_ref, we_ref, be_ref, o_ref):
    a0 = a_ref[:, 0:1]
    a1 = a_ref[:, 1:2]
    y = a0 * we_ref[0:1, :] + a1 * we_ref[1:2, :] + be_ref[...]
    o_ref[0] = y[:, :H]
    o_ref[1] = y[:, H:]

  return pl.pallas_call(
      body,
      grid=(E // BR,),
      in_specs=[
          pl.BlockSpec((BR, 2), lambda i: (i, 0)),
          pl.BlockSpec((2, _DIM), lambda i: (0, 0)),
          pl.BlockSpec((1, _DIM), lambda i: (0, 0)),
      ],
      out_specs=pl.BlockSpec((2, BR, H), lambda i: (0, i, 0)),
      out_shape=jax.ShapeDtypeStruct((2, E, H), jnp.float32),
  )(attr, We, be.reshape(1, _DIM))


_N_OBS = 50000
_N_TASK = 16384
_E1, _E2, _E3 = 500000, 400000, 160000

_E1P = _ceil_to(_E1, 32 * _EB)
_E2P = _ceil_to(_E2, 32 * _EB)
_E3P = _ceil_to(_E3, 32 * _EB)
_N1P = 50176


def _pad_edges(ei, epad, trash):
  src = ei[0].astype(jnp.int32)
  dst = ei[1].astype(jnp.int32)
  e = src.shape[0]
  src = jnp.concatenate([src, jnp.zeros((epad - e,), jnp.int32)])
  dst = jnp.concatenate([dst, jnp.full((epad - e,), trash, jnp.int32)])
  return src, dst


@functools.lru_cache(maxsize=None)
def _seg_kernels():
  seg1 = _make_seg_sum(_N_OBS, _N1P, _E1P, n_row=2)
  seg2 = _make_seg_sum(_N_TASK, _N_TASK, _E2P, n_row=1,
                       has_edge=True, E2P=_E2P)
  seg3 = _make_seg_sum(_N_TASK, _N_TASK, _E3P, n_row=1)
  return seg1, seg2, seg3


@jax.jit
def kernel(x_goal, x_obs, x_task, x_actor, edge_index_1, edge_index_2,
           edge_attr_2, edge_index_3, W1a, b1a, W2a, b2a, We, be, W1b, b1b,
           W2b, b2b, Wc1, bc1, Wc2, bc2):
  seg1, seg2, seg3 = _seg_kernels()
  src1, dst1 = _pad_edges(edge_index_1, _E1P, _N_OBS)
  src2, dst2 = _pad_edges(edge_index_2, _E2P, _N_TASK)
  src3, dst3 = _pad_edges(edge_index_3, _E3P, _N_TASK)

  o1 = seg1(x_goal.reshape(-1, _W), src1, dst1)
  agg1 = (o1[0], o1[1])
  x_obs_p = jnp.concatenate(
      [x_obs, jnp.zeros((_N1P - _N_OBS, _DIM), jnp.float32)])
  x1 = _mlp2(x_obs_p, tuple(agg1), W1a, b1a, W2a, b2a)

  attr_p = jnp.concatenate(
      [edge_attr_2, jnp.zeros((_E2P - _E2, 2), jnp.float32)])
  eemb = _edge_embed(attr_p, We, be).reshape(-1, _W)
  o2 = seg2(x1.reshape(-1, _W), eemb, src2, dst2)
  agg2 = (o2[0], o2[1])
  x2 = _mlp2(x_task, tuple(agg2), W1b, b1b, W2b, b2b)

  o3 = seg3(x2.reshape(-1, _W), src3, dst3)
  agg3 = (o3[0], o3[1])
  Wc2p = jnp.pad(Wc2, ((0, 0), (0, _DIM - Wc2.shape[1])))
  bc2p = jnp.pad(bc2, (0, _DIM - bc2.shape[0]))
  y = _mlp2(x_actor, tuple(agg3), Wc1, bc1, Wc2p, bc2p)
  return y[:, 0:1].reshape(-1, 16)


# speedup vs baseline: 1.0085x; 1.0085x over previous
import functools

import jax
import jax.numpy as jnp
from jax import lax
from jax.experimental import pallas as pl
from jax.experimental.pallas import tpu as pltpu
from jax.experimental.pallas import tpu_sc as plsc

_DIM = 128
_W = 64
_EB = 2048
_G = 128
_ZR = 64


def _ceil_to(x, m):
  return (x + m - 1) // m * m


def _make_seg_sum(N, N_out, E_pad, n_row, EB, has_edge=False, E2P=0):
  assert N % n_row == 0
  RN = N // n_row
  ET = E_pad // 16
  BLKS = ET // EB
  AR = _ceil_to(RN + 16, 128)
  SLAB = AR // 16
  WS = _ceil_to(-(-RN // 16), 8)
  LAST = RN - 15 * WS
  assert 0 < LAST <= WS and RN % 8 == 0

  mesh = plsc.VectorSubcoreMesh(core_axis_name="c", subcore_axis_name="s",
                                num_cores=2, num_subcores=16)

  assert BLKS % 2 == 0
  NC = EB // _G
  scratch = [
      pltpu.VMEM_SHARED((AR, _W), jnp.float32),
      pltpu.VMEM((EB,), jnp.int32),
      pltpu.VMEM((EB,), jnp.int32),
      pltpu.VMEM((EB,), jnp.int32),
      pltpu.VMEM((EB,), jnp.int32),
      pltpu.VMEM((EB,), jnp.int32),
      pltpu.VMEM((EB,), jnp.int32),
      pltpu.VMEM((_G, _W), jnp.float32),
      pltpu.VMEM((_G, _W), jnp.float32),
  ]
  if has_edge:
    scratch += [pltpu.VMEM((_G, _W), jnp.float32),
                pltpu.VMEM((_G, _W), jnp.float32)]
  scratch += [
      pltpu.VMEM((_ZR, _W), jnp.float32),
      pltpu.SemaphoreType.DMA,
      pltpu.SemaphoreType.DMA,
      pltpu.SemaphoreType.DMA,
      pltpu.SemaphoreType.DMA,
      pltpu.SemaphoreType.DMA,
  ]
  if has_edge:
    scratch += [pltpu.SemaphoreType.DMA,
                pltpu.SemaphoreType.DMA]

  def body(*refs):
    if has_edge:
      (table, eembr, srch, dsth, out,
       acc, sidx0, sidx1, didx0, didx1, tidx, ldix, rows0, rows1,
       erow0, erow1, zbuf, semi0, semi1, semg0, semg1, sems,
       seme0, seme1) = refs
    else:
      (table, srch, dsth, out,
       acc, sidx0, sidx1, didx0, didx1, tidx, ldix, rows0, rows1,
       zbuf, semi0, semi1, semg0, semg1, sems) = refs
      erow0 = erow1 = seme0 = seme1 = None
    sidxs = (sidx0, sidx1)
    didxs = (didx0, didx1)
    semis = (semi0, semi1)
    rowss = (rows0, rows1)
    semgs = (semg0, semg1)
    erows = (erow0, erow1)
    semes = (seme0, seme1)

    cid = lax.axis_index("c")
    sid = lax.axis_index("s")
    base_e = sid * ET

    def zfill(r, carry):
      for k in range(_W // 16):
        zbuf[r, pl.ds(k * 16, 16)] = jnp.zeros((16,), jnp.float32)
      return carry
    lax.fori_loop(0, _ZR, zfill, 0)

    for p in range(n_row):
      lo = p * RN

      off = 0
      while off < SLAB:
        sz = min(_ZR, SLAB - off)
        pltpu.sync_copy(zbuf.at[pl.ds(0, sz)],
                        acc.at[pl.ds(sid * SLAB + off, sz)])
        off += sz
      plsc.subcore_barrier()

      e0 = pl.multiple_of(base_e, EB)
      pltpu.async_copy(srch.at[pl.ds(e0, EB)], sidx0, semi0)
      pltpu.async_copy(dsth.at[pl.ds(e0, EB)], didx0, semi0)

      def scat_wait():
        pltpu.make_async_copy(rows0.at[pl.ds(0, 16)],
                              acc.at[pl.ds(0, 16)], sems).wait()

      def pbody(pb, carry):
        for par in range(2):
          sx, dx, smi = sidxs[par], didxs[par], semis[par]
          nsx, ndx, nsmi = sidxs[1 - par], didxs[1 - par], semis[1 - par]
          b = pb * 2 + par
          eoff = pl.multiple_of(base_e + b * EB, EB)
          pltpu.make_async_copy(srch.at[pl.ds(eoff, EB)], sx, smi).wait()
          pltpu.make_async_copy(dsth.at[pl.ds(eoff, EB)], dx, smi).wait()

          @pl.when(b + 1 < BLKS)
          def _():
            e2 = pl.multiple_of(eoff + EB, EB)
            pltpu.async_copy(srch.at[pl.ds(e2, EB)], nsx, nsmi)
            pltpu.async_copy(dsth.at[pl.ds(e2, EB)], ndx, nsmi)

          def sbody(i, carry):
            sv = sx[pl.ds(i * 16, 16)]
            tidx[pl.ds(i * 16, 16)] = sv * 2 + cid
            dv = dx[pl.ds(i * 16, 16)]
            if n_row == 1:
              ldix[pl.ds(i * 16, 16)] = dv
            else:
              inr = (dv >= lo) & (dv < lo + RN)
              ldix[pl.ds(i * 16, 16)] = jnp.where(
                  inr, dv - lo, jnp.int32(RN))
            return carry
          lax.fori_loop(0, EB // 16, sbody, 0)

          def fire(g):
            q = g % 2
            pltpu.async_copy(table.at[tidx.at[pl.ds(g * _G, _G)]],
                             rowss[q], semgs[q])
            if has_edge:
              pltpu.async_copy(
                  eembr.at[pl.ds(cid * E2P + eoff + g * _G, _G)],
                  erows[q], semes[q])

          fire(0)
          for g in range(NC):
            q = g % 2
            rq = rowss[q]
            if g >= 1:
              for _j in range(_G // 16):
                scat_wait()
            if g + 1 < NC:
              fire(g + 1)
            pltpu.make_async_copy(table.at[tidx.at[pl.ds(g * _G, _G)]],
                                  rq, semgs[q]).wait()
            if has_edge:
              eq = erows[q]
              pltpu.make_async_copy(
                  eembr.at[pl.ds(cid * E2P + eoff + g * _G, _G)],
                  eq, semes[q]).wait()

              def ebody(r, carry, rq=rq, eq=eq):
                for k in range(_W // 16):
                  sl = pl.ds(k * 16, 16)
                  rq[r, sl] = jnp.maximum(rq[r, sl] + eq[r, sl], 0.0)
                return carry
              lax.fori_loop(0, _G, ebody, 0)

            for j in range(_G // 16):
              idxv = ldix[pl.ds(g * _G + j * 16, 16)]
              pltpu.async_copy(rq.at[pl.ds(j * 16, 16)],
                               acc.at[idxv], sems, add=True)
          for _j in range(_G // 16):
            scat_wait()
        return carry
      lax.fori_loop(0, BLKS // 2, pbody, 0)

      plsc.subcore_barrier()
      if LAST == WS:
        pltpu.sync_copy(acc.at[pl.ds(sid * WS, WS)],
                        out.at[cid, pl.ds(lo + sid * WS, WS)])
      else:
        @pl.when(sid < 15)
        def _():
          pltpu.sync_copy(acc.at[pl.ds(sid * WS, WS)],
                          out.at[cid, pl.ds(lo + sid * WS, WS)])

        @pl.when(sid == 15)
        def _():
          pltpu.sync_copy(acc.at[pl.ds(15 * WS, LAST)],
                          out.at[cid, pl.ds(lo + 15 * WS, LAST)])
      plsc.subcore_barrier()

  out_type = jax.ShapeDtypeStruct((2, N_out, _W), jnp.float32)
  return pl.kernel(
      body, out_type=out_type, mesh=mesh, scratch_types=scratch,
      compiler_params=pltpu.CompilerParams(use_tc_tiling_on_sc=False))


def _mlp2(xa, xbs, W1, b1, W2, b2):
  N = xa.shape[0]
  BR = 1024
  assert N % BR == 0, N
  ng = len(xbs)
  W = _DIM // ng

  def body(*refs):
    xa_ref = refs[0]
    xb_refs = refs[1:1 + ng]
    w1_ref, b1_ref, w2_ref, b2_ref, o_ref = refs[1 + ng:]
    xb = jnp.concatenate([r[...] for r in xb_refs], axis=1)
    x = xa_ref[...] + xb
    h = jnp.maximum(
        jnp.dot(x, w1_ref[...], preferred_element_type=jnp.float32)
        + b1_ref[...], 0.0)
    o_ref[...] = (
        jnp.dot(h, w2_ref[...], preferred_element_type=jnp.float32)
        + b2_ref[...])

  return pl.pallas_call(
      body,
      grid=(N // BR,),
      in_specs=[pl.BlockSpec((BR, _DIM), lambda i: (i, 0))]
      + [pl.BlockSpec((BR, W), lambda i: (i, 0)) for _ in range(ng)]
      + [
          pl.BlockSpec((_DIM, _DIM), lambda i: (0, 0)),
          pl.BlockSpec((1, _DIM), lambda i: (0, 0)),
          pl.BlockSpec((_DIM, _DIM), lambda i: (0, 0)),
          pl.BlockSpec((1, _DIM), lambda i: (0, 0)),
      ],
      out_specs=pl.BlockSpec((BR, _DIM), lambda i: (i, 0)),
      out_shape=jax.ShapeDtypeStruct((N, _DIM), jnp.float32),
  )(xa, *xbs, W1, b1.reshape(1, _DIM), W2, b2.reshape(1, _DIM))


def _edge_embed(attr, We, be):
  E = attr.shape[0]
  BR = 2048
  assert E % BR == 0
  H = _DIM // 2

  def body(a_ref, we_ref, be_ref, o_ref):
    a0 = a_ref[:, 0:1]
    a1 = a_ref[:, 1:2]
    y = a0 * we_ref[0:1, :] + a1 * we_ref[1:2, :] + be_ref[...]
    o_ref[0] = y[:, :H]
    o_ref[1] = y[:, H:]

  return pl.pallas_call(
      body,
      grid=(E // BR,),
      in_specs=[
          pl.BlockSpec((BR, 2), lambda i: (i, 0)),
          pl.BlockSpec((2, _DIM), lambda i: (0, 0)),
          pl.BlockSpec((1, _DIM), lambda i: (0, 0)),
      ],
      out_specs=pl.BlockSpec((2, BR, H), lambda i: (0, i, 0)),
      out_shape=jax.ShapeDtypeStruct((2, E, H), jnp.float32),
  )(attr, We, be.reshape(1, _DIM))


_N_OBS = 50000
_N_TASK = 16384
_E1, _E2, _E3 = 500000, 400000, 160000

_E1P = _ceil_to(_E1, 32 * _EB)
_E2P = _ceil_to(_E2, 32 * _EB)
_E3P = _ceil_to(_E3, 32 * _EB)
_N1P = 50176


def _pad_edges(ei, epad, trash):
  src = ei[0].astype(jnp.int32)
  dst = ei[1].astype(jnp.int32)
  e = src.shape[0]
  src = jnp.concatenate([src, jnp.zeros((epad - e,), jnp.int32)])
  dst = jnp.concatenate([dst, jnp.full((epad - e,), trash, jnp.int32)])
  return src, dst


@functools.lru_cache(maxsize=None)
def _seg_kernels():
  seg1 = _make_seg_sum(_N_OBS, _N1P, _E1P, n_row=2, EB=1024)
  seg2 = _make_seg_sum(_N_TASK, _N_TASK, _E2P, n_row=1, EB=2048,
                       has_edge=True, E2P=_E2P)
  seg3 = _make_seg_sum(_N_TASK, _N_TASK, _E3P, n_row=1, EB=2048)
  return seg1, seg2, seg3


@jax.jit
def kernel(x_goal, x_obs, x_task, x_actor, edge_index_1, edge_index_2,
           edge_attr_2, edge_index_3, W1a, b1a, W2a, b2a, We, be, W1b, b1b,
           W2b, b2b, Wc1, bc1, Wc2, bc2):
  seg1, seg2, seg3 = _seg_kernels()
  src1, dst1 = _pad_edges(edge_index_1, _E1P, _N_OBS)
  src2, dst2 = _pad_edges(edge_index_2, _E2P, _N_TASK)
  src3, dst3 = _pad_edges(edge_index_3, _E3P, _N_TASK)

  o1 = seg1(x_goal.reshape(-1, _W), src1, dst1)
  agg1 = (o1[0], o1[1])
  x_obs_p = jnp.concatenate(
      [x_obs, jnp.zeros((_N1P - _N_OBS, _DIM), jnp.float32)])
  x1 = _mlp2(x_obs_p, tuple(agg1), W1a, b1a, W2a, b2a)

  attr_p = jnp.concatenate(
      [edge_attr_2, jnp.zeros((_E2P - _E2, 2), jnp.float32)])
  eemb = _edge_embed(attr_p, We, be).reshape(-1, _W)
  o2 = seg2(x1.reshape(-1, _W), eemb, src2, dst2)
  agg2 = (o2[0], o2[1])
  x2 = _mlp2(x_task, tuple(agg2), W1b, b1b, W2b, b2b)

  o3 = seg3(x2.reshape(-1, _W), src3, dst3)
  agg3 = (o3[0], o3[1])
  Wc2p = jnp.pad(Wc2, ((0, 0), (0, _DIM - Wc2.shape[1])))
  bc2p = jnp.pad(bc2, (0, _DIM - bc2.shape[0]))
  y = _mlp2(x_actor, tuple(agg3), Wc1, bc1, Wc2p, bc2p)
  return y[:, 0:1].reshape(-1, 16)


# speedup vs baseline: 1.0089x; 1.0004x over previous
import functools

import jax
import jax.numpy as jnp
from jax import lax
from jax.experimental import pallas as pl
from jax.experimental.pallas import tpu as pltpu
from jax.experimental.pallas import tpu_sc as plsc

_DIM = 128
_W = 64
_EB = 2048
_G = 128
_ZR = 64


def _ceil_to(x, m):
  return (x + m - 1) // m * m


def _make_seg_sum(N, N_out, E_pad, n_row, EB, has_edge=False, E2P=0):
  assert N % n_row == 0
  RN = N // n_row
  ET = E_pad // 16
  BLKS = ET // EB
  AR = _ceil_to(RN + 16, 128)
  SLAB = AR // 16
  WS = _ceil_to(-(-RN // 16), 8)
  LAST = RN - 15 * WS
  assert 0 < LAST <= WS and RN % 8 == 0

  mesh = plsc.VectorSubcoreMesh(core_axis_name="c", subcore_axis_name="s",
                                num_cores=2, num_subcores=16)

  assert BLKS % 2 == 0
  NC = EB // _G
  scratch = [
      pltpu.VMEM_SHARED((AR, _W), jnp.float32),
      pltpu.VMEM((EB,), jnp.int32),
      pltpu.VMEM((EB,), jnp.int32),
      pltpu.VMEM((EB,), jnp.int32),
      pltpu.VMEM((EB,), jnp.int32),
      pltpu.VMEM((EB,), jnp.int32),
      pltpu.VMEM((EB // _G, _G), jnp.int32),
      pltpu.VMEM((_G, _W), jnp.float32),
      pltpu.VMEM((_G, _W), jnp.float32),
  ]
  if has_edge:
    scratch += [pltpu.VMEM((_G, _W), jnp.float32),
                pltpu.VMEM((_G, _W), jnp.float32)]
  scratch += [
      pltpu.VMEM((_ZR, _W), jnp.float32),
      pltpu.SemaphoreType.DMA,
      pltpu.SemaphoreType.DMA,
      pltpu.SemaphoreType.DMA,
      pltpu.SemaphoreType.DMA,
      pltpu.SemaphoreType.DMA,
  ]
  if has_edge:
    scratch += [pltpu.SemaphoreType.DMA,
                pltpu.SemaphoreType.DMA]

  def body(*refs):
    if has_edge:
      (table, eembr, srch, dsth, out,
       acc, sidx0, sidx1, didx0, didx1, tidx, ldix, rows0, rows1,
       erow0, erow1, zbuf, semi0, semi1, semg0, semg1, sems,
       seme0, seme1) = refs
    else:
      (table, srch, dsth, out,
       acc, sidx0, sidx1, didx0, didx1, tidx, ldix, rows0, rows1,
       zbuf, semi0, semi1, semg0, semg1, sems) = refs
      erow0 = erow1 = seme0 = seme1 = None
    sidxs = (sidx0, sidx1)
    didxs = (didx0, didx1)
    semis = (semi0, semi1)
    rowss = (rows0, rows1)
    semgs = (semg0, semg1)
    erows = (erow0, erow1)
    semes = (seme0, seme1)

    cid = lax.axis_index("c")
    sid = lax.axis_index("s")
    base_e = sid * ET

    def zfill(r, carry):
      for k in range(_W // 16):
        zbuf[r, pl.ds(k * 16, 16)] = jnp.zeros((16,), jnp.float32)
      return carry
    lax.fori_loop(0, _ZR, zfill, 0)

    for p in range(n_row):
      lo = p * RN

      off = 0
      while off < SLAB:
        sz = min(_ZR, SLAB - off)
        pltpu.sync_copy(zbuf.at[pl.ds(0, sz)],
                        acc.at[pl.ds(sid * SLAB + off, sz)])
        off += sz
      plsc.subcore_barrier()

      e0 = pl.multiple_of(base_e, EB)
      pltpu.async_copy(srch.at[pl.ds(e0, EB)], sidx0, semi0)
      pltpu.async_copy(dsth.at[pl.ds(e0, EB)], didx0, semi0)

      def scat_wait(g):
        pltpu.make_async_copy(rows0, acc.at[ldix.at[g]], sems).wait()

      def pbody(pb, carry):
        for par in range(2):
          sx, dx, smi = sidxs[par], didxs[par], semis[par]
          nsx, ndx, nsmi = sidxs[1 - par], didxs[1 - par], semis[1 - par]
          b = pb * 2 + par
          eoff = pl.multiple_of(base_e + b * EB, EB)
          pltpu.make_async_copy(srch.at[pl.ds(eoff, EB)], sx, smi).wait()
          pltpu.make_async_copy(dsth.at[pl.ds(eoff, EB)], dx, smi).wait()

          @pl.when(b + 1 < BLKS)
          def _():
            e2 = pl.multiple_of(eoff + EB, EB)
            pltpu.async_copy(srch.at[pl.ds(e2, EB)], nsx, nsmi)
            pltpu.async_copy(dsth.at[pl.ds(e2, EB)], ndx, nsmi)

          def sbody(r, carry):
            for k in range(_G // 16):
              src_sl = pl.ds(r * _G + k * 16, 16)
              sv = sx[src_sl]
              tidx[src_sl] = sv * 2 + cid
              dv = dx[src_sl]
              if n_row == 1:
                ldix[r, pl.ds(k * 16, 16)] = dv
              else:
                inr = (dv >= lo) & (dv < lo + RN)
                ldix[r, pl.ds(k * 16, 16)] = jnp.where(
                    inr, dv - lo, jnp.int32(RN))
            return carry
          lax.fori_loop(0, NC, sbody, 0)

          def fire(g):
            q = g % 2
            pltpu.async_copy(table.at[tidx.at[pl.ds(g * _G, _G)]],
                             rowss[q], semgs[q])
            if has_edge:
              pltpu.async_copy(
                  eembr.at[pl.ds(cid * E2P + eoff + g * _G, _G)],
                  erows[q], semes[q])

          fire(0)
          for g in range(NC):
            q = g % 2
            rq = rowss[q]
            if g >= 1:
              scat_wait(g - 1)
            if g + 1 < NC:
              fire(g + 1)
            pltpu.make_async_copy(table.at[tidx.at[pl.ds(g * _G, _G)]],
                                  rq, semgs[q]).wait()
            if has_edge:
              eq = erows[q]
              pltpu.make_async_copy(
                  eembr.at[pl.ds(cid * E2P + eoff + g * _G, _G)],
                  eq, semes[q]).wait()

              def ebody(r, carry, rq=rq, eq=eq):
                for k in range(_W // 16):
                  sl = pl.ds(k * 16, 16)
                  rq[r, sl] = jnp.maximum(rq[r, sl] + eq[r, sl], 0.0)
                return carry
              lax.fori_loop(0, _G, ebody, 0)

            pltpu.async_copy(rq, acc.at[ldix.at[g]], sems, add=True)
          scat_wait(NC - 1)
        return carry
      lax.fori_loop(0, BLKS // 2, pbody, 0)

      plsc.subcore_barrier()
      if LAST == WS:
        pltpu.sync_copy(acc.at[pl.ds(sid * WS, WS)],
                        out.at[cid, pl.ds(lo + sid * WS, WS)])
      else:
        @pl.when(sid < 15)
        def _():
          pltpu.sync_copy(acc.at[pl.ds(sid * WS, WS)],
                          out.at[cid, pl.ds(lo + sid * WS, WS)])

        @pl.when(sid == 15)
        def _():
          pltpu.sync_copy(acc.at[pl.ds(15 * WS, LAST)],
                          out.at[cid, pl.ds(lo + 15 * WS, LAST)])
      plsc.subcore_barrier()

  out_type = jax.ShapeDtypeStruct((2, N_out, _W), jnp.float32)
  return pl.kernel(
      body, out_type=out_type, mesh=mesh, scratch_types=scratch,
      compiler_params=pltpu.CompilerParams(use_tc_tiling_on_sc=False))


def _mlp2(xa, xbs, W1, b1, W2, b2):
  N = xa.shape[0]
  BR = 1024
  assert N % BR == 0, N
  ng = len(xbs)
  W = _DIM // ng

  def body(*refs):
    xa_ref = refs[0]
    xb_refs = refs[1:1 + ng]
    w1_ref, b1_ref, w2_ref, b2_ref, o_ref = refs[1 + ng:]
    xb = jnp.concatenate([r[...] for r in xb_refs], axis=1)
    x = xa_ref[...] + xb
    h = jnp.maximum(
        jnp.dot(x, w1_ref[...], preferred_element_type=jnp.float32)
        + b1_ref[...], 0.0)
    o_ref[...] = (
        jnp.dot(h, w2_ref[...], preferred_element_type=jnp.float32)
        + b2_ref[...])

  return pl.pallas_call(
      body,
      grid=(N // BR,),
      in_specs=[pl.BlockSpec((BR, _DIM), lambda i: (i, 0))]
      + [pl.BlockSpec((BR, W), lambda i: (i, 0)) for _ in range(ng)]
      + [
          pl.BlockSpec((_DIM, _DIM), lambda i: (0, 0)),
          pl.BlockSpec((1, _DIM), lambda i: (0, 0)),
          pl.BlockSpec((_DIM, _DIM), lambda i: (0, 0)),
          pl.BlockSpec((1, _DIM), lambda i: (0, 0)),
      ],
      out_specs=pl.BlockSpec((BR, _DIM), lambda i: (i, 0)),
      out_shape=jax.ShapeDtypeStruct((N, _DIM), jnp.float32),
  )(xa, *xbs, W1, b1.reshape(1, _DIM), W2, b2.reshape(1, _DIM))


def _edge_embed(attr, We, be):
  E = attr.shape[0]
  BR = 2048
  assert E % BR == 0
  H = _DIM // 2

  def body(a_ref, we_ref, be_ref, o_ref):
    a0 = a_ref[:, 0:1]
    a1 = a_ref[:, 1:2]
    y = a0 * we_ref[0:1, :] + a1 * we_ref[1:2, :] + be_ref[...]
    o_ref[0] = y[:, :H]
    o_ref[1] = y[:, H:]

  return pl.pallas_call(
      body,
      grid=(E // BR,),
      in_specs=[
          pl.BlockSpec((BR, 2), lambda i: (i, 0)),
          pl.BlockSpec((2, _DIM), lambda i: (0, 0)),
          pl.BlockSpec((1, _DIM), lambda i: (0, 0)),
      ],
      out_specs=pl.BlockSpec((2, BR, H), lambda i: (0, i, 0)),
      out_shape=jax.ShapeDtypeStruct((2, E, H), jnp.float32),
  )(attr, We, be.reshape(1, _DIM))


_N_OBS = 50000
_N_TASK = 16384
_E1, _E2, _E3 = 500000, 400000, 160000

_E1P = _ceil_to(_E1, 32 * _EB)
_E2P = _ceil_to(_E2, 32 * _EB)
_E3P = _ceil_to(_E3, 32 * _EB)
_N1P = 50176


def _pad_edges(ei, epad, trash):
  src = ei[0].astype(jnp.int32)
  dst = ei[1].astype(jnp.int32)
  e = src.shape[0]
  src = jnp.concatenate([src, jnp.zeros((epad - e,), jnp.int32)])
  dst = jnp.concatenate([dst, jnp.full((epad - e,), trash, jnp.int32)])
  return src, dst


@functools.lru_cache(maxsize=None)
def _seg_kernels():
  seg1 = _make_seg_sum(_N_OBS, _N1P, _E1P, n_row=2, EB=1024)
  seg2 = _make_seg_sum(_N_TASK, _N_TASK, _E2P, n_row=1, EB=2048,
                       has_edge=True, E2P=_E2P)
  seg3 = _make_seg_sum(_N_TASK, _N_TASK, _E3P, n_row=1, EB=2048)
  return seg1, seg2, seg3


@jax.jit
def kernel(x_goal, x_obs, x_task, x_actor, edge_index_1, edge_index_2,
           edge_attr_2, edge_index_3, W1a, b1a, W2a, b2a, We, be, W1b, b1b,
           W2b, b2b, Wc1, bc1, Wc2, bc2):
  seg1, seg2, seg3 = _seg_kernels()
  src1, dst1 = _pad_edges(edge_index_1, _E1P, _N_OBS)
  src2, dst2 = _pad_edges(edge_index_2, _E2P, _N_TASK)
  src3, dst3 = _pad_edges(edge_index_3, _E3P, _N_TASK)

  o1 = seg1(x_goal.reshape(-1, _W), src1, dst1)
  agg1 = (o1[0], o1[1])
  x_obs_p = jnp.concatenate(
      [x_obs, jnp.zeros((_N1P - _N_OBS, _DIM), jnp.float32)])
  x1 = _mlp2(x_obs_p, tuple(agg1), W1a, b1a, W2a, b2a)

  attr_p = jnp.concatenate(
      [edge_attr_2, jnp.zeros((_E2P - _E2, 2), jnp.float32)])
  eemb = _edge_embed(attr_p, We, be).reshape(-1, _W)
  o2 = seg2(x1.reshape(-1, _W), eemb, src2, dst2)
  agg2 = (o2[0], o2[1])
  x2 = _mlp2(x_task, tuple(agg2), W1b, b1b, W2b, b2b)

  o3 = seg3(x2.reshape(-1, _W), src3, dst3)
  agg3 = (o3[0], o3[1])
  Wc2p = jnp.pad(Wc2, ((0, 0), (0, _DIM - Wc2.shape[1])))
  bc2p = jnp.pad(bc2, (0, _DIM - bc2.shape[0]))
  y = _mlp2(x_actor, tuple(agg3), Wc1, bc1, Wc2p, bc2p)
  return y[:, 0:1].reshape(-1, 16)


# speedup vs baseline: 1.0097x; 1.0008x over previous
import functools

import jax
import jax.numpy as jnp
from jax import lax
from jax.experimental import pallas as pl
from jax.experimental.pallas import tpu as pltpu
from jax.experimental.pallas import tpu_sc as plsc

_DIM = 128
_W = 64
_EB = 2048
_G = 128
_ZR = 32


def _ceil_to(x, m):
  return (x + m - 1) // m * m


def _make_seg_sum(N, N_out, E_pad, n_row, EB, G, R, K,
                  has_edge=False, E2P=0):
  assert N % n_row == 0
  RN = N // n_row
  ET = E_pad // 16
  BLKS = ET // EB
  NC = EB // G
  assert BLKS % 2 == 0 and NC >= R > K >= 1
  AR = _ceil_to(RN + 16, 128)
  SLAB = AR // 16
  WS = _ceil_to(-(-RN // 16), 8)
  LAST = RN - 15 * WS
  assert 0 < LAST <= WS and RN % 8 == 0

  mesh = plsc.VectorSubcoreMesh(core_axis_name="c", subcore_axis_name="s",
                                num_cores=2, num_subcores=16)

  scratch = [
      pltpu.VMEM_SHARED((AR, _W), jnp.float32),
      pltpu.VMEM((EB,), jnp.int32),
      pltpu.VMEM((EB,), jnp.int32),
      pltpu.VMEM((EB,), jnp.int32),
      pltpu.VMEM((EB,), jnp.int32),
      pltpu.VMEM((EB,), jnp.int32),
      pltpu.VMEM((NC, G), jnp.int32),
      pltpu.VMEM((_ZR, _W), jnp.float32),
      pltpu.SemaphoreType.DMA,
      pltpu.SemaphoreType.DMA,
  ]
  scratch += [pltpu.VMEM((G, _W), jnp.float32) for _ in range(R)]
  scratch += [pltpu.SemaphoreType.DMA for _ in range(R)]
  scratch += [pltpu.SemaphoreType.DMA for _ in range(R)]
  if has_edge:
    scratch += [pltpu.VMEM((G, _W), jnp.float32) for _ in range(R)]
    scratch += [pltpu.SemaphoreType.DMA for _ in range(R)]

  def body(*refs):
    if has_edge:
      (table, eembr, srch, dsth, out) = refs[:5]
      refs = refs[5:]
    else:
      (table, srch, dsth, out) = refs[:4]
      refs = refs[4:]
    (acc, sidx0, sidx1, didx0, didx1, tidx, ldix, zbuf,
     semi0, semi1) = refs[:10]
    refs = refs[10:]
    rowss = refs[:R]
    semgs = refs[R:2 * R]
    semss = refs[2 * R:3 * R]
    if has_edge:
      erows = refs[3 * R:4 * R]
      semes = refs[4 * R:5 * R]
    sidxs = (sidx0, sidx1)
    didxs = (didx0, didx1)
    semis = (semi0, semi1)

    cid = lax.axis_index("c")
    sid = lax.axis_index("s")
    base_e = sid * ET

    def zfill(r, carry):
      for k in range(_W // 16):
        zbuf[r, pl.ds(k * 16, 16)] = jnp.zeros((16,), jnp.float32)
      return carry
    lax.fori_loop(0, _ZR, zfill, 0)

    for p in range(n_row):
      lo = p * RN

      off = 0
      while off < SLAB:
        sz = min(_ZR, SLAB - off)
        pltpu.sync_copy(zbuf.at[pl.ds(0, sz)],
                        acc.at[pl.ds(sid * SLAB + off, sz)])
        off += sz
      plsc.subcore_barrier()

      e0 = pl.multiple_of(base_e, EB)
      pltpu.async_copy(srch.at[pl.ds(e0, EB)], sidx0, semi0)
      pltpu.async_copy(dsth.at[pl.ds(e0, EB)], didx0, semi0)

      def pbody(pb, carry):
        for par in range(2):
          sx, dx, smi = sidxs[par], didxs[par], semis[par]
          nsx, ndx, nsmi = sidxs[1 - par], didxs[1 - par], semis[1 - par]
          b = pb * 2 + par
          eoff = pl.multiple_of(base_e + b * EB, EB)
          pltpu.make_async_copy(srch.at[pl.ds(eoff, EB)], sx, smi).wait()
          pltpu.make_async_copy(dsth.at[pl.ds(eoff, EB)], dx, smi).wait()

          @pl.when(b + 1 < BLKS)
          def _():
            e2 = pl.multiple_of(eoff + EB, EB)
            pltpu.async_copy(srch.at[pl.ds(e2, EB)], nsx, nsmi)
            pltpu.async_copy(dsth.at[pl.ds(e2, EB)], ndx, nsmi)

          def sbody(r, carry):
            for k in range(G // 16):
              src_sl = pl.ds(r * G + k * 16, 16)
              sv = sx[src_sl]
              tidx[src_sl] = sv * 2 + cid
              dv = dx[src_sl]
              if n_row == 1:
                ldix[r, pl.ds(k * 16, 16)] = dv
              else:
                inr = (dv >= lo) & (dv < lo + RN)
                ldix[r, pl.ds(k * 16, 16)] = jnp.where(
                    inr, dv - lo, jnp.int32(RN))
            return carry
          lax.fori_loop(0, NC, sbody, 0)

          def fire(g):
            q = g % R
            pltpu.async_copy(table.at[tidx.at[pl.ds(g * G, G)]],
                             rowss[q], semgs[q])
            if has_edge:
              pltpu.async_copy(
                  eembr.at[pl.ds(cid * E2P + eoff + g * G, G)],
                  erows[q], semes[q])

          def scat_wait(q):
            pltpu.make_async_copy(rowss[q], acc.at[ldix.at[0]],
                                  semss[q]).wait()

          for g in range(K):
            fire(g)
          for g in range(NC):
            q = g % R
            rq = rowss[q]
            gn = g + K
            if gn < NC:
              qn = gn % R
              if gn >= R:
                scat_wait(qn)
              fire(gn)
            pltpu.make_async_copy(table.at[tidx.at[pl.ds(g * G, G)]],
                                  rq, semgs[q]).wait()
            if has_edge:
              eq = erows[q]
              pltpu.make_async_copy(
                  eembr.at[pl.ds(cid * E2P + eoff + g * G, G)],
                  eq, semes[q]).wait()

              def ebody(r, carry, rq=rq, eq=eq):
                for k in range(_W // 16):
                  sl = pl.ds(k * 16, 16)
                  rq[r, sl] = jnp.maximum(rq[r, sl] + eq[r, sl], 0.0)
                return carry
              lax.fori_loop(0, G, ebody, 0)

            pltpu.async_copy(rq, acc.at[ldix.at[g]], semss[q], add=True)
          for q in range(min(R, NC)):
            scat_wait(q)
        return carry
      lax.fori_loop(0, BLKS // 2, pbody, 0)

      plsc.subcore_barrier()
      if LAST == WS:
        pltpu.sync_copy(acc.at[pl.ds(sid * WS, WS)],
                        out.at[cid, pl.ds(lo + sid * WS, WS)])
      else:
        @pl.when(sid < 15)
        def _():
          pltpu.sync_copy(acc.at[pl.ds(sid * WS, WS)],
                          out.at[cid, pl.ds(lo + sid * WS, WS)])

        @pl.when(sid == 15)
        def _():
          pltpu.sync_copy(acc.at[pl.ds(15 * WS, LAST)],
                          out.at[cid, pl.ds(lo + 15 * WS, LAST)])
      plsc.subcore_barrier()

  out_type = jax.ShapeDtypeStruct((2, N_out, _W), jnp.float32)
  return pl.kernel(
      body, out_type=out_type, mesh=mesh, scratch_types=scratch,
      compiler_params=pltpu.CompilerParams(use_tc_tiling_on_sc=False))


def _mlp2(xa, xbs, W1, b1, W2, b2):
  N = xa.shape[0]
  BR = 1024
  assert N % BR == 0, N
  ng = len(xbs)
  W = _DIM // ng

  def body(*refs):
    xa_ref = refs[0]
    xb_refs = refs[1:1 + ng]
    w1_ref, b1_ref, w2_ref, b2_ref, o_ref = refs[1 + ng:]
    xb = jnp.concatenate([r[...] for r in xb_refs], axis=1)
    x = xa_ref[...] + xb
    h = jnp.maximum(
        jnp.dot(x, w1_ref[...], preferred_element_type=jnp.float32)
        + b1_ref[...], 0.0)
    o_ref[...] = (
        jnp.dot(h, w2_ref[...], preferred_element_type=jnp.float32)
        + b2_ref[...])

  return pl.pallas_call(
      body,
      grid=(N // BR,),
      in_specs=[pl.BlockSpec((BR, _DIM), lambda i: (i, 0))]
      + [pl.BlockSpec((BR, W), lambda i: (i, 0)) for _ in range(ng)]
      + [
          pl.BlockSpec((_DIM, _DIM), lambda i: (0, 0)),
          pl.BlockSpec((1, _DIM), lambda i: (0, 0)),
          pl.BlockSpec((_DIM, _DIM), lambda i: (0, 0)),
          pl.BlockSpec((1, _DIM), lambda i: (0, 0)),
      ],
      out_specs=pl.BlockSpec((BR, _DIM), lambda i: (i, 0)),
      out_shape=jax.ShapeDtypeStruct((N, _DIM), jnp.float32),
  )(xa, *xbs, W1, b1.reshape(1, _DIM), W2, b2.reshape(1, _DIM))


def _edge_embed(attr, We, be):
  E = attr.shape[0]
  BR = 2048
  assert E % BR == 0
  H = _DIM // 2

  def body(a_ref, we_ref, be_ref, o_ref):
    a0 = a_ref[:, 0:1]
    a1 = a_ref[:, 1:2]
    y = a0 * we_ref[0:1, :] + a1 * we_ref[1:2, :] + be_ref[...]
    o_ref[0] = y[:, :H]
    o_ref[1] = y[:, H:]

  return pl.pallas_call(
      body,
      grid=(E // BR,),
      in_specs=[
          pl.BlockSpec((BR, 2), lambda i: (i, 0)),
          pl.BlockSpec((2, _DIM), lambda i: (0, 0)),
          pl.BlockSpec((1, _DIM), lambda i: (0, 0)),
      ],
      out_specs=pl.BlockSpec((2, BR, H), lambda i: (0, i, 0)),
      out_shape=jax.ShapeDtypeStruct((2, E, H), jnp.float32),
  )(attr, We, be.reshape(1, _DIM))


_N_OBS = 50000
_N_TASK = 16384
_E1, _E2, _E3 = 500000, 400000, 160000

_E1P = _ceil_to(_E1, 32 * _EB)
_E2P = _ceil_to(_E2, 32 * _EB)
_E3P = _ceil_to(_E3, 32 * _EB)
_N1P = 50176


def _pad_edges(ei, epad, trash):
  src = ei[0].astype(jnp.int32)
  dst = ei[1].astype(jnp.int32)
  e = src.shape[0]
  src = jnp.concatenate([src, jnp.zeros((epad - e,), jnp.int32)])
  dst = jnp.concatenate([dst, jnp.full((epad - e,), trash, jnp.int32)])
  return src, dst


@functools.lru_cache(maxsize=None)
def _seg_kernels():
  seg1 = _make_seg_sum(_N_OBS, _N1P, _E1P, n_row=2,
                       EB=512, G=64, R=6, K=3)
  seg2 = _make_seg_sum(_N_TASK, _N_TASK, _E2P, n_row=1,
                       EB=2048, G=64, R=6, K=3,
                       has_edge=True, E2P=_E2P)
  seg3 = _make_seg_sum(_N_TASK, _N_TASK, _E3P, n_row=1,
                       EB=2048, G=128, R=6, K=3)
  return seg1, seg2, seg3


@jax.jit
def kernel(x_goal, x_obs, x_task, x_actor, edge_index_1, edge_index_2,
           edge_attr_2, edge_index_3, W1a, b1a, W2a, b2a, We, be, W1b, b1b,
           W2b, b2b, Wc1, bc1, Wc2, bc2):
  seg1, seg2, seg3 = _seg_kernels()
  src1, dst1 = _pad_edges(edge_index_1, _E1P, _N_OBS)
  src2, dst2 = _pad_edges(edge_index_2, _E2P, _N_TASK)
  src3, dst3 = _pad_edges(edge_index_3, _E3P, _N_TASK)

  o1 = seg1(x_goal.reshape(-1, _W), src1, dst1)
  agg1 = (o1[0], o1[1])
  x_obs_p = jnp.concatenate(
      [x_obs, jnp.zeros((_N1P - _N_OBS, _DIM), jnp.float32)])
  x1 = _mlp2(x_obs_p, tuple(agg1), W1a, b1a, W2a, b2a)

  attr_p = jnp.concatenate(
      [edge_attr_2, jnp.zeros((_E2P - _E2, 2), jnp.float32)])
  eemb = _edge_embed(attr_p, We, be).reshape(-1, _W)
  o2 = seg2(x1.reshape(-1, _W), eemb, src2, dst2)
  agg2 = (o2[0], o2[1])
  x2 = _mlp2(x_task, tuple(agg2), W1b, b1b, W2b, b2b)

  o3 = seg3(x2.reshape(-1, _W), src3, dst3)
  agg3 = (o3[0], o3[1])
  Wc2p = jnp.pad(Wc2, ((0, 0), (0, _DIM - Wc2.shape[1])))
  bc2p = jnp.pad(bc2, (0, _DIM - bc2.shape[0]))
  y = _mlp2(x_actor, tuple(agg3), Wc1, bc1, Wc2p, bc2p)
  return y[:, 0:1].reshape(-1, 16)
